# Initial kernel scaffold; baseline (speedup 1.0000x reference)
import jax
import jax.numpy as jnp
from jax.experimental import pallas as pl


def kernel(data, ent_embs, rel_embs):
    raise NotImplementedError("write your pallas kernel here")


# speedup vs baseline: 1.1043x; 1.1043x over previous
import functools

import jax
import jax.numpy as jnp
from jax import lax
from jax.experimental import pallas as pl
from jax.experimental.pallas import tpu as pltpu
from jax.experimental.pallas import tpu_sc as plsc

NUM_CORES = 2
NUM_SUBCORES = 16
NUM_WORKERS = NUM_CORES * NUM_SUBCORES
LANES = 16

BATCH = 16384
DIM = 128
CHUNK = 128
B_PER_W = BATCH // NUM_WORKERS
N_CHUNKS = B_PER_W // CHUNK


def _dist_mult_body(ent_hbm, rel_hbm, h_hbm, r_hbm, t_hbm, out_hbm,
                    idx_h, idx_r, idx_t, rows_h, rows_r, rows_t, out_v, sem):
    wid = lax.axis_index("s") * NUM_CORES + lax.axis_index("c")
    base_w = wid * B_PER_W

    for c in range(N_CHUNKS):
        base = base_w + c * CHUNK
        pltpu.sync_copy(h_hbm.at[pl.ds(base, CHUNK)], idx_h)
        pltpu.sync_copy(r_hbm.at[pl.ds(base, CHUNK)], idx_r)
        pltpu.sync_copy(t_hbm.at[pl.ds(base, CHUNK)], idx_t)

        cp_h = pltpu.make_async_copy(ent_hbm.at[idx_h], rows_h, sem)
        cp_r = pltpu.make_async_copy(rel_hbm.at[idx_r], rows_r, sem)
        cp_t = pltpu.make_async_copy(ent_hbm.at[idx_t], rows_t, sem)
        cp_h.start()
        cp_r.start()
        cp_t.start()
        cp_h.wait()
        cp_r.wait()
        cp_t.wait()

        lane_ids = lax.iota(jnp.int32, LANES)
        shuffle_idx = [lane_ids ^ s for s in (8, 4, 2, 1)]

        dnums = lax.GatherDimensionNumbers(
            offset_dims=(), collapsed_slice_dims=(0,), start_index_map=(0,))

        def lane_sum(v):
            for idx in shuffle_idx:
                v = v + lax.gather(
                    v, idx[:, None], dnums, slice_sizes=(1,),
                    mode=lax.GatherScatterMode.PROMISE_IN_BOUNDS)
            return v

        def group_body(g, _):
            res = jnp.zeros((LANES,), jnp.float32)
            for j in range(LANES):
                i = g * LANES + j
                acc = jnp.zeros((LANES,), jnp.float32)
                for k in range(DIM // LANES):
                    hv = rows_h[i, pl.ds(k * LANES, LANES)]
                    rv = rows_r[i, pl.ds(k * LANES, LANES)]
                    tv = rows_t[i, pl.ds(k * LANES, LANES)]
                    acc = acc + hv * rv * tv
                res = jnp.where(lane_ids == j, lane_sum(acc), res)
            out_v[pl.ds(g * LANES, LANES)] = jnp.clip(res, -20.0, 20.0)
            return 0

        lax.fori_loop(0, CHUNK // LANES, group_body, 0)

        pltpu.sync_copy(out_v, out_hbm.at[pl.ds(base, CHUNK)])


@jax.jit
def _dist_mult(ent_embs, rel_embs, h_idx, r_idx, t_idx):
    mesh = plsc.VectorSubcoreMesh(core_axis_name="c", subcore_axis_name="s")
    run = pl.kernel(
        _dist_mult_body,
        out_type=jax.ShapeDtypeStruct((BATCH,), jnp.float32),
        mesh=mesh,
        scratch_types=[
            pltpu.VMEM((CHUNK,), jnp.int32),
            pltpu.VMEM((CHUNK,), jnp.int32),
            pltpu.VMEM((CHUNK,), jnp.int32),
            pltpu.VMEM((CHUNK, DIM), jnp.float32),
            pltpu.VMEM((CHUNK, DIM), jnp.float32),
            pltpu.VMEM((CHUNK, DIM), jnp.float32),
            pltpu.VMEM((CHUNK,), jnp.float32),
            pltpu.SemaphoreType.DMA,
        ],
    )
    return run(ent_embs, rel_embs, h_idx, r_idx, t_idx)


def kernel(data, ent_embs, rel_embs):
    h_idx = data[:, 0].astype(jnp.int32)
    r_idx = data[:, 1].astype(jnp.int32)
    t_idx = data[:, 2].astype(jnp.int32)
    return _dist_mult(ent_embs, rel_embs, h_idx, r_idx, t_idx)


# speedup vs baseline: 2.3906x; 2.1647x over previous
import jax
import jax.numpy as jnp
from jax import lax
from jax.experimental import pallas as pl
from jax.experimental.pallas import tpu as pltpu
from jax.experimental.pallas import tpu_sc as plsc

NUM_CORES = 2
NUM_SUBCORES = 16
NUM_WORKERS = NUM_CORES * NUM_SUBCORES
LANES = 16

BATCH = 16384
DIM = 128
CHUNK = 128
B_PER_W = BATCH // NUM_WORKERS
N_CHUNKS = B_PER_W // CHUNK


def _dist_mult_body(ent_hbm, rel_hbm, h_hbm, r_hbm, t_hbm, out_hbm,
                    idx_h, idx_r, idx_t, rows_h, rows_r, rows_t, out_v,
                    sems):
    wid = lax.axis_index("s") * NUM_CORES + lax.axis_index("c")
    base_w = wid * B_PER_W

    lane_ids = lax.iota(jnp.int32, LANES)
    shuffle_idx = [lane_ids ^ s for s in (8, 4, 2, 1)]
    dnums = lax.GatherDimensionNumbers(
        offset_dims=(), collapsed_slice_dims=(0,), start_index_map=(0,))

    def lane_sum(v):
        for idx in shuffle_idx:
            v = v + lax.gather(
                v, idx[:, None], dnums, slice_sizes=(1,),
                mode=lax.GatherScatterMode.PROMISE_IN_BOUNDS)
        return v

    def start_chunk(c, buf):
        base = base_w + c * CHUNK
        pltpu.sync_copy(h_hbm.at[pl.ds(base, CHUNK)], idx_h.at[buf])
        pltpu.sync_copy(r_hbm.at[pl.ds(base, CHUNK)], idx_r.at[buf])
        pltpu.sync_copy(t_hbm.at[pl.ds(base, CHUNK)], idx_t.at[buf])
        pltpu.make_async_copy(
            ent_hbm.at[idx_h.at[buf]], rows_h.at[buf], sems.at[buf]).start()
        pltpu.make_async_copy(
            rel_hbm.at[idx_r.at[buf]], rows_r.at[buf], sems.at[buf]).start()
        pltpu.make_async_copy(
            ent_hbm.at[idx_t.at[buf]], rows_t.at[buf], sems.at[buf]).start()

    def wait_chunk(buf):
        pltpu.make_async_copy(
            ent_hbm.at[idx_h.at[buf]], rows_h.at[buf], sems.at[buf]).wait()
        pltpu.make_async_copy(
            rel_hbm.at[idx_r.at[buf]], rows_r.at[buf], sems.at[buf]).wait()
        pltpu.make_async_copy(
            ent_hbm.at[idx_t.at[buf]], rows_t.at[buf], sems.at[buf]).wait()

    def compute_chunk(c, buf):
        base = base_w + c * CHUNK
        rh = rows_h.at[buf]
        rr = rows_r.at[buf]
        rt = rows_t.at[buf]

        def group_body(g, _):
            def triple_body(j, res):
                i = g * LANES + j
                acc = jnp.zeros((LANES,), jnp.float32)
                for k in range(DIM // LANES):
                    hv = rh[i, pl.ds(k * LANES, LANES)]
                    rv = rr[i, pl.ds(k * LANES, LANES)]
                    tv = rt[i, pl.ds(k * LANES, LANES)]
                    acc = acc + hv * rv * tv
                return jnp.where(lane_ids == j, lane_sum(acc), res)

            res = lax.fori_loop(0, LANES, triple_body,
                                jnp.zeros((LANES,), jnp.float32))
            out_v[pl.ds(g * LANES, LANES)] = jnp.clip(res, -20.0, 20.0)
            return 0

        lax.fori_loop(0, CHUNK // LANES, group_body, 0)
        pltpu.sync_copy(out_v, out_hbm.at[pl.ds(base, CHUNK)])

    start_chunk(0, 0)
    for c in range(N_CHUNKS):
        buf = c % 2
        if c + 1 < N_CHUNKS:
            start_chunk(c + 1, 1 - buf)
        wait_chunk(buf)
        compute_chunk(c, buf)


@jax.jit
def _dist_mult(ent_embs, rel_embs, h_idx, r_idx, t_idx):
    mesh = plsc.VectorSubcoreMesh(core_axis_name="c", subcore_axis_name="s")
    run = pl.kernel(
        _dist_mult_body,
        out_type=jax.ShapeDtypeStruct((BATCH,), jnp.float32),
        mesh=mesh,
        scratch_types=[
            pltpu.VMEM((2, CHUNK), jnp.int32),
            pltpu.VMEM((2, CHUNK), jnp.int32),
            pltpu.VMEM((2, CHUNK), jnp.int32),
            pltpu.VMEM((2, CHUNK, DIM), jnp.float32),
            pltpu.VMEM((2, CHUNK, DIM), jnp.float32),
            pltpu.VMEM((2, CHUNK, DIM), jnp.float32),
            pltpu.VMEM((CHUNK,), jnp.float32),
            pltpu.SemaphoreType.DMA((2,)),
        ],
    )
    return run(ent_embs, rel_embs, h_idx, r_idx, t_idx)


def kernel(data, ent_embs, rel_embs):
    h_idx = data[:, 0].astype(jnp.int32)
    r_idx = data[:, 1].astype(jnp.int32)
    t_idx = data[:, 2].astype(jnp.int32)
    return _dist_mult(ent_embs, rel_embs, h_idx, r_idx, t_idx)
